# baseline (speedup 1.0000x reference)
import functools

import jax
import jax.numpy as jnp
from jax import lax
from jax.experimental import pallas as pl
from jax.experimental.pallas import tpu as pltpu
from jax.experimental.pallas import tpu_sc as plsc

E_DYN = 8
E_FIX = 1
NE = E_DYN + E_FIX
TOP_K = 2
D = 2048
DFF = 512
EPS2 = 0.02
T = 2048

M = 128
NB_SH = T // M
CAP = 2 * T + E_DYN * M
NB_DYN = CAP // M
NBT = NB_SH + NB_DYN
ROWS_ALL = T + CAP

NW = 32
TPW = T // NW
PPW = 2 * TPW

RB = 256
NRB = T // RB

NEG_INF = float("-inf")


def _silu(x):
    return x * jax.nn.sigmoid(x)


def _mixer(logits):
    scores = logits[:, :E_DYN]
    io8 = jax.lax.broadcasted_iota(jnp.int32, scores.shape, 1)

    thr1 = jnp.max(scores, axis=1, keepdims=True)
    a1 = jnp.min(jnp.where(scores == thr1, io8, E_DYN), axis=1, keepdims=True)
    factor1 = jnp.maximum(jnp.abs(scores), jnp.abs(thr1))
    m1 = (thr1 - scores) / factor1 > EPS2
    g1 = jax.nn.softmax(jnp.where(m1, NEG_INF, scores), axis=-1)
    mult1 = jnp.sum(jnp.where(io8 == a1, g1, 0.0), axis=1, keepdims=True)

    masked2 = jnp.where(io8 == a1, NEG_INF, scores)
    thr2 = jnp.max(masked2, axis=1, keepdims=True)
    a2 = jnp.min(jnp.where(masked2 == thr2, io8, E_DYN), axis=1, keepdims=True)
    factor2 = jnp.maximum(jnp.abs(scores), jnp.abs(thr2))
    m2 = (thr2 - scores) / factor2 > EPS2
    g2 = jax.nn.softmax(jnp.where(m2, NEG_INF, masked2), axis=-1)
    mult2 = jnp.sum(jnp.where(io8 == a2, g2, 0.0), axis=1, keepdims=True)

    io9 = jax.lax.broadcasted_iota(jnp.int32, logits.shape, 1)
    sel = (io9 == a1) | (io9 == a2) | (io9 == E_DYN)
    gw = jax.nn.softmax(jnp.where(sel, logits, NEG_INF), axis=-1)
    sum_gdyn = jnp.sum(gw[:, :E_DYN], axis=1, keepdims=True)
    gfix = gw[:, E_DYN:]

    return a1, a2, mult1 * sum_gdyn, mult2 * sum_gdyn, gfix


def _route_body(x_ref, wr_ref, a1_ref, a2_ref, w1_ref, w2_ref, gf_ref,
                hist_ref, sbex_ref):
    t = pl.program_id(0)
    x = x_ref[...]
    logits = jnp.dot(x, wr_ref[...], preferred_element_type=jnp.float32)
    a1, a2, w1, w2, gfix = _mixer(logits)

    a1_ref[...] = a1
    a2_ref[...] = a2
    ones16 = jnp.ones((RB, 16), jnp.float32)
    w1_ref[...] = w1 * ones16
    w2_ref[...] = w2 * ones16
    gf_ref[...] = gfix * ones16

    io8 = jax.lax.broadcasted_iota(jnp.int32, (RB, E_DYN), 1)
    cnt = (jnp.where(io8 == a1, 1.0, 0.0) + jnp.where(io8 == a2, 1.0, 0.0))
    nch = RB // TPW
    rio = jax.lax.broadcasted_iota(jnp.int32, (nch, RB), 0)
    cio = jax.lax.broadcasted_iota(jnp.int32, (nch, RB), 1)
    sel = jnp.where((cio >> 6) == rio, 1.0, 0.0)
    h = jnp.dot(sel, cnt, preferred_element_type=jnp.float32)
    hz = jnp.concatenate([h, jnp.zeros((nch, 8), jnp.float32)], axis=1)
    hist_ref[pl.ds(t * nch, nch), :] = hz.astype(jnp.int32)

    hist_f = hist_ref[...].astype(jnp.float32)
    tot = jnp.sum(hist_f, axis=0, keepdims=True)
    padded = ((tot.astype(jnp.int32) + (M - 1)) >> 7) << 7
    uio_r = jax.lax.broadcasted_iota(jnp.int32, (16, 16), 0)
    uio_c = jax.lax.broadcasted_iota(jnp.int32, (16, 16), 1)
    tri = jnp.where(uio_r <= uio_c, 1.0, 0.0)
    endv = jnp.dot(padded.astype(jnp.float32), tri,
                   preferred_element_type=jnp.float32)
    rio2 = jax.lax.broadcasted_iota(jnp.int32, (64, 16), 0)
    cio2 = jax.lax.broadcasted_iota(jnp.int32, (64, 16), 1)
    b = rio2 - NB_SH
    cmp = ((b * M).astype(jnp.float32) >= endv) & (cio2 < E_DYN) & (b >= 0)
    acc = jnp.sum(jnp.where(cmp, 1.0, 0.0), axis=1, keepdims=True)
    acc = acc.astype(jnp.int32)
    rio3 = jax.lax.broadcasted_iota(jnp.int32, (64, 1), 0)
    sbex_ref[...] = jnp.where(rio3 < NB_SH, E_DYN,
                              jnp.minimum(acc, E_DYN - 1))


@jax.jit
def _route(x, W_router):
    col_i = jax.ShapeDtypeStruct((T, 1), jnp.int32)
    row_f = jax.ShapeDtypeStruct((T, 16), jnp.float32)
    col_spec = pl.BlockSpec((RB, 1), lambda t: (t, 0))
    row_spec = pl.BlockSpec((RB, 16), lambda t: (t, 0))
    return pl.pallas_call(
        _route_body,
        grid=(NRB,),
        in_specs=[
            pl.BlockSpec((RB, D), lambda t: (t, 0)),
            pl.BlockSpec((D, NE), lambda t: (0, 0)),
        ],
        out_specs=[col_spec, col_spec, row_spec, row_spec, row_spec,
                   pl.BlockSpec((NW, 16), lambda t: (0, 0)),
                   pl.BlockSpec((64, 1), lambda t: (0, 0))],
        out_shape=[col_i, col_i, row_f, row_f, row_f,
                   jax.ShapeDtypeStruct((NW, 16), jnp.int32),
                   jax.ShapeDtypeStruct((64, 1), jnp.int32)],
        compiler_params=pltpu.CompilerParams(
            dimension_semantics=("arbitrary",),
        ),
    )(x, W_router)


def _psum_incl(vec, tmp_v, iota):
    cur = vec
    for k in (1, 2, 4, 8):
        tmp_v[...] = cur
        sh = plsc.load_gather(tmp_v, [jnp.maximum(iota - k, 0)])
        cur = cur + jnp.where(iota >= k, sh, 0)
    return cur


def _dispatch_body(a1_hbm, a2_hbm, hist_hbm, x_hbm, xs_hbm, pos_hbm,
                   a1_v, a2_v, hist_v, rows_v, tok_v, slot_v, pos_v, base_v,
                   tmp_v, sem):
    w = lax.axis_index("s") * 2 + lax.axis_index("c")
    pltpu.sync_copy(hist_hbm, hist_v)
    pltpu.sync_copy(a1_hbm.at[pl.ds(w * TPW, TPW)], a1_v)
    pltpu.sync_copy(a2_hbm.at[pl.ds(w * TPW, TPW)], a2_v)

    iota = lax.iota(jnp.int32, 16)
    zero = jnp.zeros((16,), jnp.int32)

    tot = zero
    pre = zero
    for wp in range(NW):
        row = hist_v[wp]
        tot = tot + row
        pre = pre + jnp.where(wp < w, row, 0)

    padded = ((tot + (M - 1)) >> 7) << 7
    incl = _psum_incl(padded, tmp_v, iota)
    start = incl - padded
    base_v[...] = start + pre

    for g in range(8):
        rowi = g * 8 + (iota >> 1)
        e1 = plsc.load_gather(a1_v, [rowi])
        e2 = plsc.load_gather(a2_v, [rowi])
        ev = jnp.where((iota & 1) == 0, e1, e2)
        rank = zero
        incr = zero
        for e in range(E_DYN):
            m = ev == e
            c_incl = _psum_incl(jnp.where(m, 1, 0), tmp_v, iota)
            rank = jnp.where(m, c_incl - 1, rank)
            tmp_v[...] = c_incl
            cnt = plsc.load_gather(tmp_v, [jnp.full((16,), 15, jnp.int32)])
            incr = incr + jnp.where(iota == e, cnt, 0)
        slot = plsc.load_gather(base_v, [ev]) + rank
        base_v[...] = base_v[...] + incr
        slot_v[...] = slot
        pos_v[pl.ds(g * 16, 16)] = slot + T
        tok_v[...] = w * TPW + rowi
        pltpu.async_copy(x_hbm.at[tok_v], rows_v, sem).wait()
        pltpu.async_copy(rows_v, xs_hbm.at[slot_v], sem).wait()

    pltpu.sync_copy(pos_v, pos_hbm.at[pl.ds(w * PPW, PPW)])


@jax.jit
def _dispatch(a1, a2, hist, x):
    mesh = plsc.VectorSubcoreMesh(core_axis_name="c", subcore_axis_name="s")
    f = functools.partial(
        pl.kernel,
        out_type=[
            jax.ShapeDtypeStruct((CAP, D), jnp.float32),
            jax.ShapeDtypeStruct((2 * T,), jnp.int32),
        ],
        mesh=mesh,
        scratch_types=[
            pltpu.VMEM((TPW,), jnp.int32),
            pltpu.VMEM((TPW,), jnp.int32),
            pltpu.VMEM((NW, 16), jnp.int32),
            pltpu.VMEM((16, D), jnp.float32),
            pltpu.VMEM((16,), jnp.int32),
            pltpu.VMEM((16,), jnp.int32),
            pltpu.VMEM((PPW,), jnp.int32),
            pltpu.VMEM((16,), jnp.int32),
            pltpu.VMEM((16,), jnp.int32),
            pltpu.SemaphoreType.DMA,
        ],
        compiler_params=pltpu.CompilerParams(needs_layout_passes=False),
    )(_dispatch_body)
    return f(a1, a2, hist, x)


def _gmm_body(sbex_ref, xs_ref, x_ref, wg_d, wu_d, wd_d, wg_s, wu_s, wd_s,
              out_ref):
    i = pl.program_id(0)
    e = sbex_ref[i]

    @pl.when(e == E_DYN)
    def _():
        x = x_ref[...]
        h = _silu(jnp.dot(x, wg_s[0], preferred_element_type=jnp.float32))
        h = h * jnp.dot(x, wu_s[0], preferred_element_type=jnp.float32)
        out_ref[...] = jnp.dot(h, wd_s[0], preferred_element_type=jnp.float32)

    @pl.when(e < E_DYN)
    def _():
        x = xs_ref[...]
        h = _silu(jnp.dot(x, wg_d[0], preferred_element_type=jnp.float32))
        h = h * jnp.dot(x, wu_d[0], preferred_element_type=jnp.float32)
        out_ref[...] = jnp.dot(h, wd_d[0], preferred_element_type=jnp.float32)


@jax.jit
def _gmm(sbex, xs, x, Wg_dyn, Wu_dyn, Wd_dyn, Wg_sh, Wu_sh, Wd_sh):
    clampe = lambda i, s: (jnp.minimum(s[i], E_DYN - 1), 0, 0)
    grid_spec = pltpu.PrefetchScalarGridSpec(
        num_scalar_prefetch=1,
        grid=(NBT,),
        in_specs=[
            pl.BlockSpec((M, D), lambda i, s: (jnp.maximum(i - NB_SH, 0), 0)),
            pl.BlockSpec((M, D), lambda i, s: (jnp.minimum(i, NB_SH - 1), 0)),
            pl.BlockSpec((1, D, DFF), clampe),
            pl.BlockSpec((1, D, DFF), clampe),
            pl.BlockSpec((1, DFF, D), clampe),
            pl.BlockSpec((1, D, DFF), lambda i, s: (0, 0, 0)),
            pl.BlockSpec((1, D, DFF), lambda i, s: (0, 0, 0)),
            pl.BlockSpec((1, DFF, D), lambda i, s: (0, 0, 0)),
        ],
        out_specs=pl.BlockSpec((M, D), lambda i, s: (i, 0)),
    )
    return pl.pallas_call(
        _gmm_body,
        grid_spec=grid_spec,
        out_shape=jax.ShapeDtypeStruct((ROWS_ALL, D), jnp.float32),
        compiler_params=pltpu.CompilerParams(
            dimension_semantics=("arbitrary",),
        ),
    )(sbex, xs, x, Wg_dyn, Wu_dyn, Wd_dyn, Wg_sh, Wu_sh, Wd_sh)


def _combine_body(all_hbm, pos_hbm, w1_hbm, w2_hbm, gf_hbm, out_hbm,
                  posg_v, w1_v, w2_v, gf_v, dyn_v, sh_v, out_v, sem):
    w = lax.axis_index("s") * 2 + lax.axis_index("c")
    pltpu.sync_copy(w1_hbm.at[pl.ds(w * TPW, TPW)], w1_v)
    pltpu.sync_copy(w2_hbm.at[pl.ds(w * TPW, TPW)], w2_v)
    pltpu.sync_copy(gf_hbm.at[pl.ds(w * TPW, TPW)], gf_v)

    for g in range(8):
        t0 = w * TPW + g * 8
        pltpu.sync_copy(pos_hbm.at[pl.ds(w * PPW + g * 16, 16)], posg_v)
        pltpu.async_copy(all_hbm.at[posg_v], dyn_v, sem).wait()
        pltpu.sync_copy(all_hbm.at[pl.ds(t0, 8)], sh_v)
        for j in range(8):
            w0 = w1_v[g * 8 + j]
            w1 = w2_v[g * 8 + j]
            gf = gf_v[g * 8 + j]

            def vbody(v, _):
                sl = pl.ds(v * 16, 16)
                out_v[j, sl] = (w0 * dyn_v[2 * j, sl] + w1 * dyn_v[2 * j + 1, sl]
                                + gf * sh_v[j, sl])
                return _

            lax.fori_loop(0, D // 16, vbody, None)
        pltpu.sync_copy(out_v, out_hbm.at[pl.ds(t0, 8)])


@jax.jit
def _combine(allout, pos, w1, w2, gf):
    mesh = plsc.VectorSubcoreMesh(core_axis_name="c", subcore_axis_name="s")
    f = functools.partial(
        pl.kernel,
        out_type=jax.ShapeDtypeStruct((T, D), jnp.float32),
        mesh=mesh,
        scratch_types=[
            pltpu.VMEM((16,), jnp.int32),
            pltpu.VMEM((TPW, 16), jnp.float32),
            pltpu.VMEM((TPW, 16), jnp.float32),
            pltpu.VMEM((TPW, 16), jnp.float32),
            pltpu.VMEM((16, D), jnp.float32),
            pltpu.VMEM((8, D), jnp.float32),
            pltpu.VMEM((8, D), jnp.float32),
            pltpu.SemaphoreType.DMA,
        ],
        compiler_params=pltpu.CompilerParams(needs_layout_passes=False),
    )(_combine_body)
    return f(allout, pos, w1, w2, gf)


def kernel(hidden_states, W_router, Wg_dyn, Wu_dyn, Wd_dyn, Wg_sh, Wu_sh, Wd_sh):
    B, S, Dm = hidden_states.shape
    x = hidden_states.reshape(-1, Dm)
    a1, a2, w1, w2, gf, hist, sbex = _route(x, W_router)
    a1, a2 = a1.reshape(T), a2.reshape(T)
    sbex = sbex.reshape(64)
    xs, pos = _dispatch(a1, a2, hist, x)
    allout = _gmm(sbex, xs, x, Wg_dyn, Wu_dyn, Wd_dyn, Wg_sh, Wu_sh, Wd_sh)
    out = _combine(allout, pos, w1, w2, gf)
    return out.reshape(B, S, Dm)


# speedup vs baseline: 1.1076x; 1.1076x over previous
import functools

import jax
import jax.numpy as jnp
from jax import lax
from jax.experimental import pallas as pl
from jax.experimental.pallas import tpu as pltpu
from jax.experimental.pallas import tpu_sc as plsc

E_DYN = 8
E_FIX = 1
NE = E_DYN + E_FIX
TOP_K = 2
D = 2048
DFF = 512
EPS2 = 0.02
T = 2048

M = 128
NB_SH = T // M
CAP = 2 * T + E_DYN * M
NB_DYN = CAP // M
NBT = NB_SH + NB_DYN
ROWS_ALL = T + CAP

NW = 32
TPW = T // NW
PPW = 2 * TPW

RB = 256
NRB = T // RB

NEG_INF = float("-inf")


def _silu(x):
    return x * jax.nn.sigmoid(x)


def _mixer(logits):
    scores = logits[:, :E_DYN]
    io8 = jax.lax.broadcasted_iota(jnp.int32, scores.shape, 1)

    thr1 = jnp.max(scores, axis=1, keepdims=True)
    a1 = jnp.min(jnp.where(scores == thr1, io8, E_DYN), axis=1, keepdims=True)
    factor1 = jnp.maximum(jnp.abs(scores), jnp.abs(thr1))
    m1 = (thr1 - scores) / factor1 > EPS2
    g1 = jax.nn.softmax(jnp.where(m1, NEG_INF, scores), axis=-1)
    mult1 = jnp.sum(jnp.where(io8 == a1, g1, 0.0), axis=1, keepdims=True)

    masked2 = jnp.where(io8 == a1, NEG_INF, scores)
    thr2 = jnp.max(masked2, axis=1, keepdims=True)
    a2 = jnp.min(jnp.where(masked2 == thr2, io8, E_DYN), axis=1, keepdims=True)
    factor2 = jnp.maximum(jnp.abs(scores), jnp.abs(thr2))
    m2 = (thr2 - scores) / factor2 > EPS2
    g2 = jax.nn.softmax(jnp.where(m2, NEG_INF, masked2), axis=-1)
    mult2 = jnp.sum(jnp.where(io8 == a2, g2, 0.0), axis=1, keepdims=True)

    io9 = jax.lax.broadcasted_iota(jnp.int32, logits.shape, 1)
    sel = (io9 == a1) | (io9 == a2) | (io9 == E_DYN)
    gw = jax.nn.softmax(jnp.where(sel, logits, NEG_INF), axis=-1)
    sum_gdyn = jnp.sum(gw[:, :E_DYN], axis=1, keepdims=True)
    gfix = gw[:, E_DYN:]

    return a1, a2, mult1 * sum_gdyn, mult2 * sum_gdyn, gfix


def _route_body(x_ref, wr_ref, a1_ref, a2_ref, w1_ref, w2_ref, gf_ref,
                hist_ref, sbex_ref):
    t = pl.program_id(0)
    x = x_ref[...]
    logits = jnp.dot(x, wr_ref[...], preferred_element_type=jnp.float32)
    a1, a2, w1, w2, gfix = _mixer(logits)

    a1_ref[...] = a1
    a2_ref[...] = a2
    ones16 = jnp.ones((RB, 16), jnp.float32)
    w1_ref[...] = w1 * ones16
    w2_ref[...] = w2 * ones16
    gf_ref[...] = gfix * ones16

    io8 = jax.lax.broadcasted_iota(jnp.int32, (RB, E_DYN), 1)
    cnt = (jnp.where(io8 == a1, 1.0, 0.0) + jnp.where(io8 == a2, 1.0, 0.0))
    nch = RB // TPW
    rio = jax.lax.broadcasted_iota(jnp.int32, (nch, RB), 0)
    cio = jax.lax.broadcasted_iota(jnp.int32, (nch, RB), 1)
    sel = jnp.where((cio >> 6) == rio, 1.0, 0.0)
    h = jnp.dot(sel, cnt, preferred_element_type=jnp.float32)
    hz = jnp.concatenate([h, jnp.zeros((nch, 8), jnp.float32)], axis=1)
    hist_ref[pl.ds(t * nch, nch), :] = hz.astype(jnp.int32)

    hist_f = hist_ref[...].astype(jnp.float32)
    tot = jnp.sum(hist_f, axis=0, keepdims=True)
    padded = ((tot.astype(jnp.int32) + (M - 1)) >> 7) << 7
    uio_r = jax.lax.broadcasted_iota(jnp.int32, (16, 16), 0)
    uio_c = jax.lax.broadcasted_iota(jnp.int32, (16, 16), 1)
    tri = jnp.where(uio_r <= uio_c, 1.0, 0.0)
    endv = jnp.dot(padded.astype(jnp.float32), tri,
                   preferred_element_type=jnp.float32)
    rio2 = jax.lax.broadcasted_iota(jnp.int32, (64, 16), 0)
    cio2 = jax.lax.broadcasted_iota(jnp.int32, (64, 16), 1)
    b = rio2 - NB_SH
    cmp = ((b * M).astype(jnp.float32) >= endv) & (cio2 < E_DYN) & (b >= 0)
    acc = jnp.sum(jnp.where(cmp, 1.0, 0.0), axis=1, keepdims=True)
    acc = acc.astype(jnp.int32)
    rio3 = jax.lax.broadcasted_iota(jnp.int32, (64, 1), 0)
    sbex_ref[...] = jnp.where(rio3 < NB_SH, E_DYN,
                              jnp.minimum(acc, E_DYN - 1))


@jax.jit
def _route(x, W_router):
    col_i = jax.ShapeDtypeStruct((T, 1), jnp.int32)
    row_f = jax.ShapeDtypeStruct((T, 16), jnp.float32)
    col_spec = pl.BlockSpec((RB, 1), lambda t: (t, 0))
    row_spec = pl.BlockSpec((RB, 16), lambda t: (t, 0))
    return pl.pallas_call(
        _route_body,
        grid=(NRB,),
        in_specs=[
            pl.BlockSpec((RB, D), lambda t: (t, 0)),
            pl.BlockSpec((D, NE), lambda t: (0, 0)),
        ],
        out_specs=[col_spec, col_spec, row_spec, row_spec, row_spec,
                   pl.BlockSpec((NW, 16), lambda t: (0, 0)),
                   pl.BlockSpec((64, 1), lambda t: (0, 0))],
        out_shape=[col_i, col_i, row_f, row_f, row_f,
                   jax.ShapeDtypeStruct((NW, 16), jnp.int32),
                   jax.ShapeDtypeStruct((64, 1), jnp.int32)],
        compiler_params=pltpu.CompilerParams(
            dimension_semantics=("arbitrary",),
        ),
    )(x, W_router)


def _psum_incl(vec, tmp_v, iota):
    cur = vec
    for k in (1, 2, 4, 8):
        tmp_v[...] = cur
        sh = plsc.load_gather(tmp_v, [jnp.maximum(iota - k, 0)])
        cur = cur + jnp.where(iota >= k, sh, 0)
    return cur


def _dispatch_body(a1_hbm, a2_hbm, hist_hbm, x_hbm, xs_hbm, pos_hbm,
                   a1_v, a2_v, hist_v, rows_v, tok_v, slot_v, pos_v, base_v,
                   tmp_v, sem):
    w = lax.axis_index("s") * 2 + lax.axis_index("c")
    pltpu.sync_copy(hist_hbm, hist_v)
    pltpu.sync_copy(a1_hbm.at[pl.ds(w * TPW, TPW)], a1_v)
    pltpu.sync_copy(a2_hbm.at[pl.ds(w * TPW, TPW)], a2_v)

    iota = lax.iota(jnp.int32, 16)
    zero = jnp.zeros((16,), jnp.int32)

    tot = zero
    pre = zero
    for wp in range(NW):
        row = hist_v[wp]
        tot = tot + row
        pre = pre + jnp.where(wp < w, row, 0)

    padded = ((tot + (M - 1)) >> 7) << 7
    incl = _psum_incl(padded, tmp_v, iota)
    start = incl - padded
    base_v[...] = start + pre

    for g in range(8):
        rowi = g * 8 + (iota >> 1)
        e1 = plsc.load_gather(a1_v, [rowi])
        e2 = plsc.load_gather(a2_v, [rowi])
        ev = jnp.where((iota & 1) == 0, e1, e2)
        rank = zero
        incr = zero
        for e in range(E_DYN):
            m = ev == e
            c_incl = _psum_incl(jnp.where(m, 1, 0), tmp_v, iota)
            rank = jnp.where(m, c_incl - 1, rank)
            tmp_v[...] = c_incl
            cnt = plsc.load_gather(tmp_v, [jnp.full((16,), 15, jnp.int32)])
            incr = incr + jnp.where(iota == e, cnt, 0)
        slot = plsc.load_gather(base_v, [ev]) + rank
        base_v[...] = base_v[...] + incr
        slot_v[...] = slot
        pos_v[pl.ds(g * 16, 16)] = slot + T
        tok_v[...] = w * TPW + rowi
        pltpu.async_copy(x_hbm.at[tok_v], rows_v, sem).wait()
        pltpu.async_copy(rows_v, xs_hbm.at[slot_v], sem).wait()

    pltpu.sync_copy(pos_v, pos_hbm.at[pl.ds(w * PPW, PPW)])


@jax.jit
def _dispatch(a1, a2, hist, x):
    mesh = plsc.VectorSubcoreMesh(core_axis_name="c", subcore_axis_name="s")
    f = functools.partial(
        pl.kernel,
        out_type=[
            jax.ShapeDtypeStruct((CAP, D), jnp.float32),
            jax.ShapeDtypeStruct((2 * T,), jnp.int32),
        ],
        mesh=mesh,
        scratch_types=[
            pltpu.VMEM((TPW,), jnp.int32),
            pltpu.VMEM((TPW,), jnp.int32),
            pltpu.VMEM((NW, 16), jnp.int32),
            pltpu.VMEM((16, D), jnp.float32),
            pltpu.VMEM((16,), jnp.int32),
            pltpu.VMEM((16,), jnp.int32),
            pltpu.VMEM((PPW,), jnp.int32),
            pltpu.VMEM((16,), jnp.int32),
            pltpu.VMEM((16,), jnp.int32),
            pltpu.SemaphoreType.DMA,
        ],
        compiler_params=pltpu.CompilerParams(needs_layout_passes=False),
    )(_dispatch_body)
    return f(a1, a2, hist, x)


def _gmm_body(sbex_ref, xs_ref, x_ref, wg_d, wu_d, wd_d, wg_s, wu_s, wd_s,
              out_ref):
    i = pl.program_id(0)
    e = sbex_ref[i]

    @pl.when(e == E_DYN)
    def _():
        x = x_ref[...]
        h = _silu(jnp.dot(x, wg_s[0], preferred_element_type=jnp.float32))
        h = h * jnp.dot(x, wu_s[0], preferred_element_type=jnp.float32)
        out_ref[...] = jnp.dot(h, wd_s[0], preferred_element_type=jnp.float32)

    @pl.when(e < E_DYN)
    def _():
        x = xs_ref[...]
        h = _silu(jnp.dot(x, wg_d[0], preferred_element_type=jnp.float32))
        h = h * jnp.dot(x, wu_d[0], preferred_element_type=jnp.float32)
        out_ref[...] = jnp.dot(h, wd_d[0], preferred_element_type=jnp.float32)


@jax.jit
def _gmm(sbex, xs, x, Wg_dyn, Wu_dyn, Wd_dyn, Wg_sh, Wu_sh, Wd_sh):
    clampe = lambda i, s: (jnp.minimum(s[i], E_DYN - 1), 0, 0)
    grid_spec = pltpu.PrefetchScalarGridSpec(
        num_scalar_prefetch=1,
        grid=(NBT,),
        in_specs=[
            pl.BlockSpec((M, D), lambda i, s: (jnp.maximum(i - NB_SH, 0), 0)),
            pl.BlockSpec((M, D), lambda i, s: (jnp.minimum(i, NB_SH - 1), 0)),
            pl.BlockSpec((1, D, DFF), clampe),
            pl.BlockSpec((1, D, DFF), clampe),
            pl.BlockSpec((1, DFF, D), clampe),
            pl.BlockSpec((1, D, DFF), lambda i, s: (0, 0, 0)),
            pl.BlockSpec((1, D, DFF), lambda i, s: (0, 0, 0)),
            pl.BlockSpec((1, DFF, D), lambda i, s: (0, 0, 0)),
        ],
        out_specs=pl.BlockSpec((M, D), lambda i, s: (i, 0)),
    )
    return pl.pallas_call(
        _gmm_body,
        grid_spec=grid_spec,
        out_shape=jax.ShapeDtypeStruct((ROWS_ALL, D), jnp.float32),
        compiler_params=pltpu.CompilerParams(
            dimension_semantics=("arbitrary",),
        ),
    )(sbex, xs, x, Wg_dyn, Wu_dyn, Wd_dyn, Wg_sh, Wu_sh, Wd_sh)


def _combine_body(all_hbm, pos_hbm, w1_hbm, w2_hbm, gf_hbm, out_hbm,
                  posg_v, w1_v, w2_v, gf_v, dyn_v, sh_v, out_v,
                  sga, sgb, soa, sob):
    w = lax.axis_index("s") * 2 + lax.axis_index("c")
    pltpu.sync_copy(w1_hbm.at[pl.ds(w * TPW, TPW)], w1_v)
    pltpu.sync_copy(w2_hbm.at[pl.ds(w * TPW, TPW)], w2_v)
    pltpu.sync_copy(gf_hbm.at[pl.ds(w * TPW, TPW)], gf_v)
    gsem = (sga, sgb)
    osem = (soa, sob)

    def fire(g, b):
        pltpu.sync_copy(pos_hbm.at[pl.ds(w * PPW + g * 16, 16)],
                        posg_v.at[b])
        return pltpu.async_copy(all_hbm.at[posg_v.at[b]], dyn_v.at[b],
                                gsem[b])

    handles = {0: fire(0, 0)}
    out_h = [None, None]
    for g in range(8):
        b = g % 2
        t0 = w * TPW + g * 8
        if g < 7:
            handles[g + 1] = fire(g + 1, 1 - b)
        handles[g].wait()
        pltpu.sync_copy(all_hbm.at[pl.ds(t0, 8)], sh_v)
        if out_h[0] is not None:
            out_h[0].wait()
        for j in range(8):
            w0 = w1_v[g * 8 + j]
            w1 = w2_v[g * 8 + j]
            gf = gf_v[g * 8 + j]

            def vbody(vo, _):
                for vi in range(8):
                    sl = pl.ds(vo * 128 + vi * 16, 16)
                    out_v[j, sl] = (w0 * dyn_v[b, 2 * j, sl]
                                    + w1 * dyn_v[b, 2 * j + 1, sl]
                                    + gf * sh_v[j, sl])
                return _

            lax.fori_loop(0, D // 128, vbody, None)
        out_h[0] = pltpu.async_copy(out_v, out_hbm.at[pl.ds(t0, 8)],
                                    osem[0])
    out_h[0].wait()


@jax.jit
def _combine(allout, pos, w1, w2, gf):
    mesh = plsc.VectorSubcoreMesh(core_axis_name="c", subcore_axis_name="s")
    f = functools.partial(
        pl.kernel,
        out_type=jax.ShapeDtypeStruct((T, D), jnp.float32),
        mesh=mesh,
        scratch_types=[
            pltpu.VMEM((2, 16), jnp.int32),
            pltpu.VMEM((TPW, 16), jnp.float32),
            pltpu.VMEM((TPW, 16), jnp.float32),
            pltpu.VMEM((TPW, 16), jnp.float32),
            pltpu.VMEM((2, 16, D), jnp.float32),
            pltpu.VMEM((8, D), jnp.float32),
            pltpu.VMEM((8, D), jnp.float32),
            pltpu.SemaphoreType.DMA,
            pltpu.SemaphoreType.DMA,
            pltpu.SemaphoreType.DMA,
            pltpu.SemaphoreType.DMA,
        ],
        compiler_params=pltpu.CompilerParams(needs_layout_passes=False),
    )(_combine_body)
    return f(allout, pos, w1, w2, gf)


def kernel(hidden_states, W_router, Wg_dyn, Wu_dyn, Wd_dyn, Wg_sh, Wu_sh, Wd_sh):
    B, S, Dm = hidden_states.shape
    x = hidden_states.reshape(-1, Dm)
    a1, a2, w1, w2, gf, hist, sbex = _route(x, W_router)
    a1, a2 = a1.reshape(T), a2.reshape(T)
    sbex = sbex.reshape(64)
    xs, pos = _dispatch(a1, a2, hist, x)
    allout = _gmm(sbex, xs, x, Wg_dyn, Wu_dyn, Wd_dyn, Wg_sh, Wu_sh, Wd_sh)
    out = _combine(allout, pos, w1, w2, gf)
    return out.reshape(B, S, Dm)


# speedup vs baseline: 1.1465x; 1.0351x over previous
import functools

import jax
import jax.numpy as jnp
from jax import lax
from jax.experimental import pallas as pl
from jax.experimental.pallas import tpu as pltpu
from jax.experimental.pallas import tpu_sc as plsc

E_DYN = 8
E_FIX = 1
NE = E_DYN + E_FIX
TOP_K = 2
D = 2048
DFF = 512
EPS2 = 0.02
T = 2048

M = 256
LOG2M = 8
NB_SH = T // M
CAP = 2 * T + E_DYN * M
NB_DYN = CAP // M
NBT = NB_SH + NB_DYN
ROWS_ALL = T + CAP

NW = 32
TPW = T // NW
PPW = 2 * TPW

RB = 256
NRB = T // RB

NEG_INF = float("-inf")


def _silu(x):
    return x * jax.nn.sigmoid(x)


def _mixer(logits):
    scores = logits[:, :E_DYN]
    io8 = jax.lax.broadcasted_iota(jnp.int32, scores.shape, 1)

    thr1 = jnp.max(scores, axis=1, keepdims=True)
    a1 = jnp.min(jnp.where(scores == thr1, io8, E_DYN), axis=1, keepdims=True)
    factor1 = jnp.maximum(jnp.abs(scores), jnp.abs(thr1))
    m1 = (thr1 - scores) / factor1 > EPS2
    g1 = jax.nn.softmax(jnp.where(m1, NEG_INF, scores), axis=-1)
    mult1 = jnp.sum(jnp.where(io8 == a1, g1, 0.0), axis=1, keepdims=True)

    masked2 = jnp.where(io8 == a1, NEG_INF, scores)
    thr2 = jnp.max(masked2, axis=1, keepdims=True)
    a2 = jnp.min(jnp.where(masked2 == thr2, io8, E_DYN), axis=1, keepdims=True)
    factor2 = jnp.maximum(jnp.abs(scores), jnp.abs(thr2))
    m2 = (thr2 - scores) / factor2 > EPS2
    g2 = jax.nn.softmax(jnp.where(m2, NEG_INF, masked2), axis=-1)
    mult2 = jnp.sum(jnp.where(io8 == a2, g2, 0.0), axis=1, keepdims=True)

    io9 = jax.lax.broadcasted_iota(jnp.int32, logits.shape, 1)
    sel = (io9 == a1) | (io9 == a2) | (io9 == E_DYN)
    gw = jax.nn.softmax(jnp.where(sel, logits, NEG_INF), axis=-1)
    sum_gdyn = jnp.sum(gw[:, :E_DYN], axis=1, keepdims=True)
    gfix = gw[:, E_DYN:]

    return a1, a2, mult1 * sum_gdyn, mult2 * sum_gdyn, gfix


def _route_body(x_ref, wr_ref, a1_ref, a2_ref, w1_ref, w2_ref, gf_ref,
                hist_ref, sbex_ref):
    t = pl.program_id(0)
    x = x_ref[...]
    logits = jnp.dot(x, wr_ref[...], preferred_element_type=jnp.float32)
    a1, a2, w1, w2, gfix = _mixer(logits)

    a1_ref[...] = a1
    a2_ref[...] = a2
    ones16 = jnp.ones((RB, 16), jnp.float32)
    w1_ref[...] = w1 * ones16
    w2_ref[...] = w2 * ones16
    gf_ref[...] = gfix * ones16

    io8 = jax.lax.broadcasted_iota(jnp.int32, (RB, E_DYN), 1)
    cnt = (jnp.where(io8 == a1, 1.0, 0.0) + jnp.where(io8 == a2, 1.0, 0.0))
    nch = RB // TPW
    rio = jax.lax.broadcasted_iota(jnp.int32, (nch, RB), 0)
    cio = jax.lax.broadcasted_iota(jnp.int32, (nch, RB), 1)
    sel = jnp.where((cio >> 6) == rio, 1.0, 0.0)
    h = jnp.dot(sel, cnt, preferred_element_type=jnp.float32)
    hz = jnp.concatenate([h, jnp.zeros((nch, 8), jnp.float32)], axis=1)
    hist_ref[pl.ds(t * nch, nch), :] = hz.astype(jnp.int32)

    hist_f = hist_ref[...].astype(jnp.float32)
    tot = jnp.sum(hist_f, axis=0, keepdims=True)
    padded = ((tot.astype(jnp.int32) + (M - 1)) >> LOG2M) << LOG2M
    uio_r = jax.lax.broadcasted_iota(jnp.int32, (16, 16), 0)
    uio_c = jax.lax.broadcasted_iota(jnp.int32, (16, 16), 1)
    tri = jnp.where(uio_r <= uio_c, 1.0, 0.0)
    endv = jnp.dot(padded.astype(jnp.float32), tri,
                   preferred_element_type=jnp.float32)
    rio2 = jax.lax.broadcasted_iota(jnp.int32, (64, 16), 0)
    cio2 = jax.lax.broadcasted_iota(jnp.int32, (64, 16), 1)
    b = rio2 - NB_SH
    cmp = ((b * M).astype(jnp.float32) >= endv) & (cio2 < E_DYN) & (b >= 0)
    acc = jnp.sum(jnp.where(cmp, 1.0, 0.0), axis=1, keepdims=True)
    acc = acc.astype(jnp.int32)
    rio3 = jax.lax.broadcasted_iota(jnp.int32, (64, 1), 0)
    sbex_ref[...] = jnp.where(rio3 < NB_SH, E_DYN,
                              jnp.minimum(acc, E_DYN - 1))


@jax.jit
def _route(x, W_router):
    col_i = jax.ShapeDtypeStruct((T, 1), jnp.int32)
    row_f = jax.ShapeDtypeStruct((T, 16), jnp.float32)
    col_spec = pl.BlockSpec((RB, 1), lambda t: (t, 0))
    row_spec = pl.BlockSpec((RB, 16), lambda t: (t, 0))
    return pl.pallas_call(
        _route_body,
        grid=(NRB,),
        in_specs=[
            pl.BlockSpec((RB, D), lambda t: (t, 0)),
            pl.BlockSpec((D, NE), lambda t: (0, 0)),
        ],
        out_specs=[col_spec, col_spec, row_spec, row_spec, row_spec,
                   pl.BlockSpec((NW, 16), lambda t: (0, 0)),
                   pl.BlockSpec((64, 1), lambda t: (0, 0))],
        out_shape=[col_i, col_i, row_f, row_f, row_f,
                   jax.ShapeDtypeStruct((NW, 16), jnp.int32),
                   jax.ShapeDtypeStruct((64, 1), jnp.int32)],
        compiler_params=pltpu.CompilerParams(
            dimension_semantics=("arbitrary",),
        ),
    )(x, W_router)


def _psum_incl(vec, tmp_v, iota):
    cur = vec
    for k in (1, 2, 4, 8):
        tmp_v[...] = cur
        sh = plsc.load_gather(tmp_v, [jnp.maximum(iota - k, 0)])
        cur = cur + jnp.where(iota >= k, sh, 0)
    return cur


def _dispatch_body(a1_hbm, a2_hbm, hist_hbm, x_hbm, xs_hbm, pos_hbm,
                   a1_v, a2_v, hist_v, rows_v, tok_v, slot_v, pos_v, base_v,
                   tmp_v, sem):
    w = lax.axis_index("s") * 2 + lax.axis_index("c")
    pltpu.sync_copy(hist_hbm, hist_v)
    pltpu.sync_copy(a1_hbm.at[pl.ds(w * TPW, TPW)], a1_v)
    pltpu.sync_copy(a2_hbm.at[pl.ds(w * TPW, TPW)], a2_v)

    iota = lax.iota(jnp.int32, 16)
    zero = jnp.zeros((16,), jnp.int32)

    tot = zero
    pre = zero
    for wp in range(NW):
        row = hist_v[wp]
        tot = tot + row
        pre = pre + jnp.where(wp < w, row, 0)

    padded = ((tot + (M - 1)) >> LOG2M) << LOG2M
    incl = _psum_incl(padded, tmp_v, iota)
    start = incl - padded
    base_v[...] = start + pre

    for g in range(8):
        rowi = g * 8 + (iota >> 1)
        e1 = plsc.load_gather(a1_v, [rowi])
        e2 = plsc.load_gather(a2_v, [rowi])
        ev = jnp.where((iota & 1) == 0, e1, e2)
        rank = zero
        incr = zero
        for e in range(E_DYN):
            m = ev == e
            c_incl = _psum_incl(jnp.where(m, 1, 0), tmp_v, iota)
            rank = jnp.where(m, c_incl - 1, rank)
            tmp_v[...] = c_incl
            cnt = plsc.load_gather(tmp_v, [jnp.full((16,), 15, jnp.int32)])
            incr = incr + jnp.where(iota == e, cnt, 0)
        slot = plsc.load_gather(base_v, [ev]) + rank
        base_v[...] = base_v[...] + incr
        slot_v[...] = slot
        pos_v[pl.ds(g * 16, 16)] = slot + T
        tok_v[...] = w * TPW + rowi
        pltpu.async_copy(x_hbm.at[tok_v], rows_v, sem).wait()
        pltpu.async_copy(rows_v, xs_hbm.at[slot_v], sem).wait()

    pltpu.sync_copy(pos_v, pos_hbm.at[pl.ds(w * PPW, PPW)])


@jax.jit
def _dispatch(a1, a2, hist, x):
    mesh = plsc.VectorSubcoreMesh(core_axis_name="c", subcore_axis_name="s")
    f = functools.partial(
        pl.kernel,
        out_type=[
            jax.ShapeDtypeStruct((CAP, D), jnp.float32),
            jax.ShapeDtypeStruct((2 * T,), jnp.int32),
        ],
        mesh=mesh,
        scratch_types=[
            pltpu.VMEM((TPW,), jnp.int32),
            pltpu.VMEM((TPW,), jnp.int32),
            pltpu.VMEM((NW, 16), jnp.int32),
            pltpu.VMEM((16, D), jnp.float32),
            pltpu.VMEM((16,), jnp.int32),
            pltpu.VMEM((16,), jnp.int32),
            pltpu.VMEM((PPW,), jnp.int32),
            pltpu.VMEM((16,), jnp.int32),
            pltpu.VMEM((16,), jnp.int32),
            pltpu.SemaphoreType.DMA,
        ],
        compiler_params=pltpu.CompilerParams(needs_layout_passes=False),
    )(_dispatch_body)
    return f(a1, a2, hist, x)


def _gmm_body(sbex_ref, xs_ref, x_ref, wg_d, wu_d, wd_d, wg_s, wu_s, wd_s,
              out_ref):
    i = pl.program_id(0)
    e = sbex_ref[i]

    @pl.when(e == E_DYN)
    def _():
        x = x_ref[...]
        h = _silu(jnp.dot(x, wg_s[0], preferred_element_type=jnp.float32))
        h = h * jnp.dot(x, wu_s[0], preferred_element_type=jnp.float32)
        out_ref[...] = jnp.dot(h, wd_s[0], preferred_element_type=jnp.float32)

    @pl.when(e < E_DYN)
    def _():
        x = xs_ref[...]
        h = _silu(jnp.dot(x, wg_d[0], preferred_element_type=jnp.float32))
        h = h * jnp.dot(x, wu_d[0], preferred_element_type=jnp.float32)
        out_ref[...] = jnp.dot(h, wd_d[0], preferred_element_type=jnp.float32)


@jax.jit
def _gmm(sbex, xs, x, Wg_dyn, Wu_dyn, Wd_dyn, Wg_sh, Wu_sh, Wd_sh):
    clampe = lambda i, s: (jnp.minimum(s[i], E_DYN - 1), 0, 0)
    grid_spec = pltpu.PrefetchScalarGridSpec(
        num_scalar_prefetch=1,
        grid=(NBT,),
        in_specs=[
            pl.BlockSpec((M, D), lambda i, s: (jnp.maximum(i - NB_SH, 0), 0)),
            pl.BlockSpec((M, D), lambda i, s: (jnp.minimum(i, NB_SH - 1), 0)),
            pl.BlockSpec((1, D, DFF), clampe),
            pl.BlockSpec((1, D, DFF), clampe),
            pl.BlockSpec((1, DFF, D), clampe),
            pl.BlockSpec((1, D, DFF), lambda i, s: (0, 0, 0)),
            pl.BlockSpec((1, D, DFF), lambda i, s: (0, 0, 0)),
            pl.BlockSpec((1, DFF, D), lambda i, s: (0, 0, 0)),
        ],
        out_specs=pl.BlockSpec((M, D), lambda i, s: (i, 0)),
    )
    return pl.pallas_call(
        _gmm_body,
        grid_spec=grid_spec,
        out_shape=jax.ShapeDtypeStruct((ROWS_ALL, D), jnp.float32),
        compiler_params=pltpu.CompilerParams(
            dimension_semantics=("arbitrary",),
        ),
    )(sbex, xs, x, Wg_dyn, Wu_dyn, Wd_dyn, Wg_sh, Wu_sh, Wd_sh)


def _combine_body(all_hbm, pos_hbm, w1_hbm, w2_hbm, gf_hbm, out_hbm,
                  posg_v, w1_v, w2_v, gf_v, dyn_v, sh_v, out_v,
                  sga, sgb, soa, sob):
    w = lax.axis_index("s") * 2 + lax.axis_index("c")
    pltpu.sync_copy(w1_hbm.at[pl.ds(w * TPW, TPW)], w1_v)
    pltpu.sync_copy(w2_hbm.at[pl.ds(w * TPW, TPW)], w2_v)
    pltpu.sync_copy(gf_hbm.at[pl.ds(w * TPW, TPW)], gf_v)
    gsem = (sga, sgb)
    osem = (soa, sob)

    def fire(g, b):
        pltpu.sync_copy(pos_hbm.at[pl.ds(w * PPW + g * 16, 16)],
                        posg_v.at[b])
        return pltpu.async_copy(all_hbm.at[posg_v.at[b]], dyn_v.at[b],
                                gsem[b])

    handles = {0: fire(0, 0)}
    out_h = [None, None]
    for g in range(8):
        b = g % 2
        t0 = w * TPW + g * 8
        if g < 7:
            handles[g + 1] = fire(g + 1, 1 - b)
        handles[g].wait()
        pltpu.sync_copy(all_hbm.at[pl.ds(t0, 8)], sh_v)
        if out_h[0] is not None:
            out_h[0].wait()
        for j in range(8):
            w0 = w1_v[g * 8 + j]
            w1 = w2_v[g * 8 + j]
            gf = gf_v[g * 8 + j]

            def vbody(vo, _):
                for vi in range(8):
                    sl = pl.ds(vo * 128 + vi * 16, 16)
                    out_v[j, sl] = (w0 * dyn_v[b, 2 * j, sl]
                                    + w1 * dyn_v[b, 2 * j + 1, sl]
                                    + gf * sh_v[j, sl])
                return _

            lax.fori_loop(0, D // 128, vbody, None)
        out_h[0] = pltpu.async_copy(out_v, out_hbm.at[pl.ds(t0, 8)],
                                    osem[0])
    out_h[0].wait()


@jax.jit
def _combine(allout, pos, w1, w2, gf):
    mesh = plsc.VectorSubcoreMesh(core_axis_name="c", subcore_axis_name="s")
    f = functools.partial(
        pl.kernel,
        out_type=jax.ShapeDtypeStruct((T, D), jnp.float32),
        mesh=mesh,
        scratch_types=[
            pltpu.VMEM((2, 16), jnp.int32),
            pltpu.VMEM((TPW, 16), jnp.float32),
            pltpu.VMEM((TPW, 16), jnp.float32),
            pltpu.VMEM((TPW, 16), jnp.float32),
            pltpu.VMEM((2, 16, D), jnp.float32),
            pltpu.VMEM((8, D), jnp.float32),
            pltpu.VMEM((8, D), jnp.float32),
            pltpu.SemaphoreType.DMA,
            pltpu.SemaphoreType.DMA,
            pltpu.SemaphoreType.DMA,
            pltpu.SemaphoreType.DMA,
        ],
        compiler_params=pltpu.CompilerParams(needs_layout_passes=False),
    )(_combine_body)
    return f(allout, pos, w1, w2, gf)


def kernel(hidden_states, W_router, Wg_dyn, Wu_dyn, Wd_dyn, Wg_sh, Wu_sh, Wd_sh):
    B, S, Dm = hidden_states.shape
    x = hidden_states.reshape(-1, Dm)
    a1, a2, w1, w2, gf, hist, sbex = _route(x, W_router)
    a1, a2 = a1.reshape(T), a2.reshape(T)
    sbex = sbex.reshape(64)
    xs, pos = _dispatch(a1, a2, hist, x)
    allout = _gmm(sbex, xs, x, Wg_dyn, Wu_dyn, Wd_dyn, Wg_sh, Wu_sh, Wd_sh)
    out = _combine(allout, pos, w1, w2, gf)
    return out.reshape(B, S, Dm)


# speedup vs baseline: 1.1799x; 1.0292x over previous
import functools

import jax
import jax.numpy as jnp
from jax import lax
from jax.experimental import pallas as pl
from jax.experimental.pallas import tpu as pltpu
from jax.experimental.pallas import tpu_sc as plsc

E_DYN = 8
E_FIX = 1
NE = E_DYN + E_FIX
TOP_K = 2
D = 2048
DFF = 512
EPS2 = 0.02
T = 2048

M = 256
LOG2M = 8
NB_SH = T // M
CAP = 2 * T + E_DYN * M
NB_DYN = CAP // M
NBT = NB_SH + NB_DYN
ROWS_ALL = T + CAP

NW = 32
TPW = T // NW
PPW = 2 * TPW

RB = 256
NRB = T // RB

NEG_INF = float("-inf")


def _silu(x):
    return x * jax.nn.sigmoid(x)


def _mixer(logits):
    scores = logits[:, :E_DYN]
    io8 = jax.lax.broadcasted_iota(jnp.int32, scores.shape, 1)

    thr1 = jnp.max(scores, axis=1, keepdims=True)
    a1 = jnp.min(jnp.where(scores == thr1, io8, E_DYN), axis=1, keepdims=True)
    factor1 = jnp.maximum(jnp.abs(scores), jnp.abs(thr1))
    m1 = (thr1 - scores) / factor1 > EPS2
    g1 = jax.nn.softmax(jnp.where(m1, NEG_INF, scores), axis=-1)
    mult1 = jnp.sum(jnp.where(io8 == a1, g1, 0.0), axis=1, keepdims=True)

    masked2 = jnp.where(io8 == a1, NEG_INF, scores)
    thr2 = jnp.max(masked2, axis=1, keepdims=True)
    a2 = jnp.min(jnp.where(masked2 == thr2, io8, E_DYN), axis=1, keepdims=True)
    factor2 = jnp.maximum(jnp.abs(scores), jnp.abs(thr2))
    m2 = (thr2 - scores) / factor2 > EPS2
    g2 = jax.nn.softmax(jnp.where(m2, NEG_INF, masked2), axis=-1)
    mult2 = jnp.sum(jnp.where(io8 == a2, g2, 0.0), axis=1, keepdims=True)

    io9 = jax.lax.broadcasted_iota(jnp.int32, logits.shape, 1)
    sel = (io9 == a1) | (io9 == a2) | (io9 == E_DYN)
    gw = jax.nn.softmax(jnp.where(sel, logits, NEG_INF), axis=-1)
    sum_gdyn = jnp.sum(gw[:, :E_DYN], axis=1, keepdims=True)
    gfix = gw[:, E_DYN:]

    return a1, a2, mult1 * sum_gdyn, mult2 * sum_gdyn, gfix


def _route_body(x_ref, wr_ref, a1_ref, a2_ref, w1_ref, w2_ref, gf_ref,
                hist_ref, sbex_ref):
    t = pl.program_id(0)
    x = x_ref[...]
    logits = jnp.dot(x, wr_ref[...], preferred_element_type=jnp.float32)
    a1, a2, w1, w2, gfix = _mixer(logits)

    a1_ref[...] = a1
    a2_ref[...] = a2
    ones16 = jnp.ones((RB, 16), jnp.float32)
    w1_ref[...] = w1 * ones16
    w2_ref[...] = w2 * ones16
    gf_ref[...] = gfix * ones16

    io8 = jax.lax.broadcasted_iota(jnp.int32, (RB, E_DYN), 1)
    cnt = (jnp.where(io8 == a1, 1.0, 0.0) + jnp.where(io8 == a2, 1.0, 0.0))
    nch = RB // TPW
    rio = jax.lax.broadcasted_iota(jnp.int32, (nch, RB), 0)
    cio = jax.lax.broadcasted_iota(jnp.int32, (nch, RB), 1)
    sel = jnp.where((cio >> 6) == rio, 1.0, 0.0)
    h = jnp.dot(sel, cnt, preferred_element_type=jnp.float32)
    hz = jnp.concatenate([h, jnp.zeros((nch, 8), jnp.float32)], axis=1)
    hist_ref[pl.ds(t * nch, nch), :] = hz.astype(jnp.int32)

    @pl.when(t == NRB - 1)
    def _():
        _sbex_from_hist(hist_ref, sbex_ref)


def _sbex_from_hist(hist_ref, sbex_ref):
    hist_f = hist_ref[...].astype(jnp.float32)
    tot = jnp.sum(hist_f, axis=0, keepdims=True)
    padded = ((tot.astype(jnp.int32) + (M - 1)) >> LOG2M) << LOG2M
    uio_r = jax.lax.broadcasted_iota(jnp.int32, (16, 16), 0)
    uio_c = jax.lax.broadcasted_iota(jnp.int32, (16, 16), 1)
    tri = jnp.where(uio_r <= uio_c, 1.0, 0.0)
    endv = jnp.dot(padded.astype(jnp.float32), tri,
                   preferred_element_type=jnp.float32)
    rio2 = jax.lax.broadcasted_iota(jnp.int32, (64, 16), 0)
    cio2 = jax.lax.broadcasted_iota(jnp.int32, (64, 16), 1)
    b = rio2 - NB_SH
    cmp = ((b * M).astype(jnp.float32) >= endv) & (cio2 < E_DYN) & (b >= 0)
    acc = jnp.sum(jnp.where(cmp, 1.0, 0.0), axis=1, keepdims=True)
    acc = acc.astype(jnp.int32)
    rio3 = jax.lax.broadcasted_iota(jnp.int32, (64, 1), 0)
    sbex_ref[...] = jnp.where(rio3 < NB_SH, E_DYN,
                              jnp.minimum(acc, E_DYN - 1))


@jax.jit
def _route(x, W_router):
    col_i = jax.ShapeDtypeStruct((T, 1), jnp.int32)
    row_f = jax.ShapeDtypeStruct((T, 16), jnp.float32)
    col_spec = pl.BlockSpec((RB, 1), lambda t: (t, 0))
    row_spec = pl.BlockSpec((RB, 16), lambda t: (t, 0))
    return pl.pallas_call(
        _route_body,
        grid=(NRB,),
        in_specs=[
            pl.BlockSpec((RB, D), lambda t: (t, 0)),
            pl.BlockSpec((D, NE), lambda t: (0, 0)),
        ],
        out_specs=[col_spec, col_spec, row_spec, row_spec, row_spec,
                   pl.BlockSpec((NW, 16), lambda t: (0, 0)),
                   pl.BlockSpec((64, 1), lambda t: (0, 0))],
        out_shape=[col_i, col_i, row_f, row_f, row_f,
                   jax.ShapeDtypeStruct((NW, 16), jnp.int32),
                   jax.ShapeDtypeStruct((64, 1), jnp.int32)],
        compiler_params=pltpu.CompilerParams(
            dimension_semantics=("arbitrary",),
        ),
    )(x, W_router)


def _psum_incl(vec, tmp_v, iota):
    cur = vec
    for k in (1, 2, 4, 8):
        tmp_v[...] = cur
        sh = plsc.load_gather(tmp_v, [jnp.maximum(iota - k, 0)])
        cur = cur + jnp.where(iota >= k, sh, 0)
    return cur


def _dispatch_body(a1_hbm, a2_hbm, hist_hbm, x_hbm, xs_hbm, pos_hbm,
                   a1_v, a2_v, hist_v, rows_v, tok_v, slot_v, pos_v, base_v,
                   tmp_v, sga, sgb, ssa, ssb):
    w = lax.axis_index("s") * 2 + lax.axis_index("c")
    pltpu.sync_copy(hist_hbm, hist_v)
    pltpu.sync_copy(a1_hbm.at[pl.ds(w * TPW, TPW)], a1_v)
    pltpu.sync_copy(a2_hbm.at[pl.ds(w * TPW, TPW)], a2_v)

    iota = lax.iota(jnp.int32, 16)
    zero = jnp.zeros((16,), jnp.int32)

    tot = zero
    pre = zero
    for wp in range(NW):
        row = hist_v[wp]
        tot = tot + row
        pre = pre + jnp.where(wp < w, row, 0)

    padded = ((tot + (M - 1)) >> LOG2M) << LOG2M
    incl = _psum_incl(padded, tmp_v, iota)
    start = incl - padded
    base_v[...] = start + pre

    def compute(g, b):
        rowi = g * 8 + (iota >> 1)
        e1 = plsc.load_gather(a1_v, [rowi])
        e2 = plsc.load_gather(a2_v, [rowi])
        ev = jnp.where((iota & 1) == 0, e1, e2)
        rank = zero
        incr = zero
        for e in range(E_DYN):
            m = ev == e
            c_incl = _psum_incl(jnp.where(m, 1, 0), tmp_v, iota)
            rank = jnp.where(m, c_incl - 1, rank)
            tmp_v[...] = c_incl
            cnt = plsc.load_gather(tmp_v, [jnp.full((16,), 15, jnp.int32)])
            incr = incr + jnp.where(iota == e, cnt, 0)
        slot = plsc.load_gather(base_v, [ev]) + rank
        base_v[...] = base_v[...] + incr
        slot_v[b] = slot
        pos_v[pl.ds(g * 16, 16)] = slot + T
        tok_v[b] = w * TPW + rowi

    gsem = (sga, sgb)
    ssem = (ssa, ssb)
    compute(0, 0)
    gh = {0: pltpu.async_copy(x_hbm.at[tok_v.at[0]], rows_v.at[0], gsem[0])}
    sh = [None, None]
    for g in range(8):
        b = g & 1
        if g < 7:
            if sh[1 - b] is not None:
                sh[1 - b].wait()
            compute(g + 1, 1 - b)
            gh[g + 1] = pltpu.async_copy(x_hbm.at[tok_v.at[1 - b]],
                                         rows_v.at[1 - b], gsem[1 - b])
        gh[g].wait()
        sh[b] = pltpu.async_copy(rows_v.at[b], xs_hbm.at[slot_v.at[b]],
                                 ssem[b])
    sh[0].wait()
    sh[1].wait()

    pltpu.sync_copy(pos_v, pos_hbm.at[pl.ds(w * PPW, PPW)])


@jax.jit
def _dispatch(a1, a2, hist, x):
    mesh = plsc.VectorSubcoreMesh(core_axis_name="c", subcore_axis_name="s")
    f = functools.partial(
        pl.kernel,
        out_type=[
            jax.ShapeDtypeStruct((CAP, D), jnp.float32),
            jax.ShapeDtypeStruct((2 * T,), jnp.int32),
        ],
        mesh=mesh,
        scratch_types=[
            pltpu.VMEM((TPW,), jnp.int32),
            pltpu.VMEM((TPW,), jnp.int32),
            pltpu.VMEM((NW, 16), jnp.int32),
            pltpu.VMEM((2, 16, D), jnp.float32),
            pltpu.VMEM((2, 16), jnp.int32),
            pltpu.VMEM((2, 16), jnp.int32),
            pltpu.VMEM((PPW,), jnp.int32),
            pltpu.VMEM((16,), jnp.int32),
            pltpu.VMEM((16,), jnp.int32),
            pltpu.SemaphoreType.DMA,
            pltpu.SemaphoreType.DMA,
            pltpu.SemaphoreType.DMA,
            pltpu.SemaphoreType.DMA,
        ],
        compiler_params=pltpu.CompilerParams(needs_layout_passes=False),
    )(_dispatch_body)
    return f(a1, a2, hist, x)


def _gmm_body(sbex_ref, xs_ref, x_ref, wg_d, wu_d, wd_d, wg_s, wu_s, wd_s,
              out_ref):
    i = pl.program_id(0)
    e = sbex_ref[i]

    @pl.when(e == E_DYN)
    def _():
        x = x_ref[...]
        h = _silu(jnp.dot(x, wg_s[0], preferred_element_type=jnp.float32))
        h = h * jnp.dot(x, wu_s[0], preferred_element_type=jnp.float32)
        out_ref[...] = jnp.dot(h, wd_s[0], preferred_element_type=jnp.float32)

    @pl.when(e < E_DYN)
    def _():
        x = xs_ref[...]
        h = _silu(jnp.dot(x, wg_d[0], preferred_element_type=jnp.float32))
        h = h * jnp.dot(x, wu_d[0], preferred_element_type=jnp.float32)
        out_ref[...] = jnp.dot(h, wd_d[0], preferred_element_type=jnp.float32)


@jax.jit
def _gmm(sbex, xs, x, Wg_dyn, Wu_dyn, Wd_dyn, Wg_sh, Wu_sh, Wd_sh):
    clampe = lambda i, s: (jnp.minimum(s[i], E_DYN - 1), 0, 0)
    grid_spec = pltpu.PrefetchScalarGridSpec(
        num_scalar_prefetch=1,
        grid=(NBT,),
        in_specs=[
            pl.BlockSpec((M, D), lambda i, s: (jnp.maximum(i - NB_SH, 0), 0)),
            pl.BlockSpec((M, D), lambda i, s: (jnp.minimum(i, NB_SH - 1), 0)),
            pl.BlockSpec((1, D, DFF), clampe),
            pl.BlockSpec((1, D, DFF), clampe),
            pl.BlockSpec((1, DFF, D), clampe),
            pl.BlockSpec((1, D, DFF), lambda i, s: (0, 0, 0)),
            pl.BlockSpec((1, D, DFF), lambda i, s: (0, 0, 0)),
            pl.BlockSpec((1, DFF, D), lambda i, s: (0, 0, 0)),
        ],
        out_specs=pl.BlockSpec((M, D), lambda i, s: (i, 0)),
    )
    return pl.pallas_call(
        _gmm_body,
        grid_spec=grid_spec,
        out_shape=jax.ShapeDtypeStruct((ROWS_ALL, D), jnp.float32),
        compiler_params=pltpu.CompilerParams(
            dimension_semantics=("arbitrary",),
        ),
    )(sbex, xs, x, Wg_dyn, Wu_dyn, Wd_dyn, Wg_sh, Wu_sh, Wd_sh)


def _combine_body(all_hbm, pos_hbm, w1_hbm, w2_hbm, gf_hbm, out_hbm,
                  posg_v, w1_v, w2_v, gf_v, dyn_v, sh_v, out_v,
                  sga, sgb, soa, sob):
    w = lax.axis_index("s") * 2 + lax.axis_index("c")
    pltpu.sync_copy(w1_hbm.at[pl.ds(w * TPW, TPW)], w1_v)
    pltpu.sync_copy(w2_hbm.at[pl.ds(w * TPW, TPW)], w2_v)
    pltpu.sync_copy(gf_hbm.at[pl.ds(w * TPW, TPW)], gf_v)
    gsem = (sga, sgb)
    osem = (soa, sob)

    def fire(g, b):
        pltpu.sync_copy(pos_hbm.at[pl.ds(w * PPW + g * 16, 16)],
                        posg_v.at[b])
        return pltpu.async_copy(all_hbm.at[posg_v.at[b]], dyn_v.at[b],
                                gsem[b])

    handles = {0: fire(0, 0)}
    out_h = [None, None]
    for g in range(8):
        b = g % 2
        t0 = w * TPW + g * 8
        if g < 7:
            handles[g + 1] = fire(g + 1, 1 - b)
        handles[g].wait()
        pltpu.sync_copy(all_hbm.at[pl.ds(t0, 8)], sh_v)
        if out_h[0] is not None:
            out_h[0].wait()
        for j in range(8):
            w0 = w1_v[g * 8 + j]
            w1 = w2_v[g * 8 + j]
            gf = gf_v[g * 8 + j]

            def vbody(vo, _):
                for vi in range(8):
                    sl = pl.ds(vo * 128 + vi * 16, 16)
                    out_v[j, sl] = (w0 * dyn_v[b, 2 * j, sl]
                                    + w1 * dyn_v[b, 2 * j + 1, sl]
                                    + gf * sh_v[j, sl])
                return _

            lax.fori_loop(0, D // 128, vbody, None)
        out_h[0] = pltpu.async_copy(out_v, out_hbm.at[pl.ds(t0, 8)],
                                    osem[0])
    out_h[0].wait()


@jax.jit
def _combine(allout, pos, w1, w2, gf):
    mesh = plsc.VectorSubcoreMesh(core_axis_name="c", subcore_axis_name="s")
    f = functools.partial(
        pl.kernel,
        out_type=jax.ShapeDtypeStruct((T, D), jnp.float32),
        mesh=mesh,
        scratch_types=[
            pltpu.VMEM((2, 16), jnp.int32),
            pltpu.VMEM((TPW, 16), jnp.float32),
            pltpu.VMEM((TPW, 16), jnp.float32),
            pltpu.VMEM((TPW, 16), jnp.float32),
            pltpu.VMEM((2, 16, D), jnp.float32),
            pltpu.VMEM((8, D), jnp.float32),
            pltpu.VMEM((8, D), jnp.float32),
            pltpu.SemaphoreType.DMA,
            pltpu.SemaphoreType.DMA,
            pltpu.SemaphoreType.DMA,
            pltpu.SemaphoreType.DMA,
        ],
        compiler_params=pltpu.CompilerParams(needs_layout_passes=False),
    )(_combine_body)
    return f(allout, pos, w1, w2, gf)


def kernel(hidden_states, W_router, Wg_dyn, Wu_dyn, Wd_dyn, Wg_sh, Wu_sh, Wd_sh):
    B, S, Dm = hidden_states.shape
    x = hidden_states.reshape(-1, Dm)
    a1, a2, w1, w2, gf, hist, sbex = _route(x, W_router)
    a1, a2 = a1.reshape(T), a2.reshape(T)
    sbex = sbex.reshape(64)
    xs, pos = _dispatch(a1, a2, hist, x)
    allout = _gmm(sbex, xs, x, Wg_dyn, Wu_dyn, Wd_dyn, Wg_sh, Wu_sh, Wd_sh)
    out = _combine(allout, pos, w1, w2, gf)
    return out.reshape(B, S, Dm)


# speedup vs baseline: 1.2158x; 1.0305x over previous
import functools

import jax
import jax.numpy as jnp
from jax import lax
from jax.experimental import pallas as pl
from jax.experimental.pallas import tpu as pltpu
from jax.experimental.pallas import tpu_sc as plsc

E_DYN = 8
E_FIX = 1
NE = E_DYN + E_FIX
TOP_K = 2
D = 2048
DFF = 512
EPS2 = 0.02
T = 2048

M = 256
LOG2M = 8
NB_SH = T // M
CAP = 2 * T + E_DYN * M
NB_DYN = CAP // M
NBT = NB_SH + NB_DYN
ROWS_ALL = T + CAP

NW = 32
TPW = T // NW
PPW = 2 * TPW

RB = 256
NRB = T // RB

NEG_INF = float("-inf")


def _silu(x):
    return x * jax.nn.sigmoid(x)


def _mixer(logits):
    scores = logits[:, :E_DYN]
    io8 = jax.lax.broadcasted_iota(jnp.int32, scores.shape, 1)

    thr1 = jnp.max(scores, axis=1, keepdims=True)
    a1 = jnp.min(jnp.where(scores == thr1, io8, E_DYN), axis=1, keepdims=True)
    factor1 = jnp.maximum(jnp.abs(scores), jnp.abs(thr1))
    m1 = (thr1 - scores) / factor1 > EPS2
    g1 = jax.nn.softmax(jnp.where(m1, NEG_INF, scores), axis=-1)
    mult1 = jnp.sum(jnp.where(io8 == a1, g1, 0.0), axis=1, keepdims=True)

    masked2 = jnp.where(io8 == a1, NEG_INF, scores)
    thr2 = jnp.max(masked2, axis=1, keepdims=True)
    a2 = jnp.min(jnp.where(masked2 == thr2, io8, E_DYN), axis=1, keepdims=True)
    factor2 = jnp.maximum(jnp.abs(scores), jnp.abs(thr2))
    m2 = (thr2 - scores) / factor2 > EPS2
    g2 = jax.nn.softmax(jnp.where(m2, NEG_INF, masked2), axis=-1)
    mult2 = jnp.sum(jnp.where(io8 == a2, g2, 0.0), axis=1, keepdims=True)

    io9 = jax.lax.broadcasted_iota(jnp.int32, logits.shape, 1)
    sel = (io9 == a1) | (io9 == a2) | (io9 == E_DYN)
    gw = jax.nn.softmax(jnp.where(sel, logits, NEG_INF), axis=-1)
    sum_gdyn = jnp.sum(gw[:, :E_DYN], axis=1, keepdims=True)
    gfix = gw[:, E_DYN:]

    return a1, a2, mult1 * sum_gdyn, mult2 * sum_gdyn, gfix


def _route_body(x_ref, wr_ref, a1_ref, a2_ref, w1_ref, w2_ref, gf_ref,
                hist_ref, sbex_ref):
    t = pl.program_id(0)
    x = x_ref[...]
    logits = jnp.dot(x, wr_ref[...], preferred_element_type=jnp.float32)
    a1, a2, w1, w2, gfix = _mixer(logits)

    a1_ref[...] = a1
    a2_ref[...] = a2
    ones16 = jnp.ones((RB, 16), jnp.float32)
    w1_ref[...] = w1 * ones16
    w2_ref[...] = w2 * ones16
    gf_ref[...] = gfix * ones16

    io8 = jax.lax.broadcasted_iota(jnp.int32, (RB, E_DYN), 1)
    cnt = (jnp.where(io8 == a1, 1.0, 0.0) + jnp.where(io8 == a2, 1.0, 0.0))
    nch = RB // TPW
    rio = jax.lax.broadcasted_iota(jnp.int32, (nch, RB), 0)
    cio = jax.lax.broadcasted_iota(jnp.int32, (nch, RB), 1)
    sel = jnp.where((cio >> 6) == rio, 1.0, 0.0)
    h = jnp.dot(sel, cnt, preferred_element_type=jnp.float32)
    hz = jnp.concatenate([h, jnp.zeros((nch, 8), jnp.float32)], axis=1)
    hist_ref[pl.ds(t * nch, nch), :] = hz.astype(jnp.int32)

    @pl.when(t == NRB - 1)
    def _():
        _sbex_from_hist(hist_ref, sbex_ref)


def _sbex_from_hist(hist_ref, sbex_ref):
    hist_f = hist_ref[...].astype(jnp.float32)
    tot = jnp.sum(hist_f, axis=0, keepdims=True)
    padded = ((tot.astype(jnp.int32) + (M - 1)) >> LOG2M) << LOG2M
    uio_r = jax.lax.broadcasted_iota(jnp.int32, (16, 16), 0)
    uio_c = jax.lax.broadcasted_iota(jnp.int32, (16, 16), 1)
    tri = jnp.where(uio_r <= uio_c, 1.0, 0.0)
    endv = jnp.dot(padded.astype(jnp.float32), tri,
                   preferred_element_type=jnp.float32)
    rio2 = jax.lax.broadcasted_iota(jnp.int32, (64, 16), 0)
    cio2 = jax.lax.broadcasted_iota(jnp.int32, (64, 16), 1)
    b = rio2 - NB_SH
    cmp = ((b * M).astype(jnp.float32) >= endv) & (cio2 < E_DYN) & (b >= 0)
    acc = jnp.sum(jnp.where(cmp, 1.0, 0.0), axis=1, keepdims=True)
    acc = acc.astype(jnp.int32)
    rio3 = jax.lax.broadcasted_iota(jnp.int32, (64, 1), 0)
    sbex_ref[...] = jnp.where(rio3 < NB_SH, E_DYN,
                              jnp.minimum(acc, E_DYN - 1))


@jax.jit
def _route(x, W_router):
    col_i = jax.ShapeDtypeStruct((T, 1), jnp.int32)
    row_f = jax.ShapeDtypeStruct((T, 16), jnp.float32)
    col_spec = pl.BlockSpec((RB, 1), lambda t: (t, 0))
    row_spec = pl.BlockSpec((RB, 16), lambda t: (t, 0))
    return pl.pallas_call(
        _route_body,
        grid=(NRB,),
        in_specs=[
            pl.BlockSpec((RB, D), lambda t: (t, 0)),
            pl.BlockSpec((D, NE), lambda t: (0, 0)),
        ],
        out_specs=[col_spec, col_spec, row_spec, row_spec, row_spec,
                   pl.BlockSpec((NW, 16), lambda t: (0, 0)),
                   pl.BlockSpec((64, 1), lambda t: (0, 0))],
        out_shape=[col_i, col_i, row_f, row_f, row_f,
                   jax.ShapeDtypeStruct((NW, 16), jnp.int32),
                   jax.ShapeDtypeStruct((64, 1), jnp.int32)],
        compiler_params=pltpu.CompilerParams(
            dimension_semantics=("arbitrary",),
        ),
    )(x, W_router)


def _psum_incl(vec, tmp_v, iota):
    cur = vec
    for k in (1, 2, 4, 8):
        tmp_v[...] = cur
        sh = plsc.load_gather(tmp_v, [jnp.maximum(iota - k, 0)])
        cur = cur + jnp.where(iota >= k, sh, 0)
    return cur


def _dispatch_body(a1_hbm, a2_hbm, hist_hbm, x_hbm, xs_hbm, pos_hbm,
                   a1_v, a2_v, hist_v, rows_v, tok_v, slot_v, pos_v, base_v,
                   tmp_v, sga, sgb, ssa, ssb):
    w = lax.axis_index("s") * 2 + lax.axis_index("c")
    pltpu.sync_copy(hist_hbm, hist_v)
    pltpu.sync_copy(a1_hbm.at[pl.ds(w * TPW, TPW)], a1_v)
    pltpu.sync_copy(a2_hbm.at[pl.ds(w * TPW, TPW)], a2_v)

    iota = lax.iota(jnp.int32, 16)
    zero = jnp.zeros((16,), jnp.int32)

    tot = zero
    pre = zero
    for wp in range(NW):
        row = hist_v[wp]
        tot = tot + row
        pre = pre + jnp.where(wp < w, row, 0)

    padded = ((tot + (M - 1)) >> LOG2M) << LOG2M
    incl = _psum_incl(padded, tmp_v, iota)
    start = incl - padded
    base_v[...] = start + pre

    def compute(g, b):
        rowi = g * 8 + (iota >> 1)
        e1 = plsc.load_gather(a1_v, [rowi])
        e2 = plsc.load_gather(a2_v, [rowi])
        ev = jnp.where((iota & 1) == 0, e1, e2)
        rank = zero
        incr = zero
        for e in range(E_DYN):
            m = ev == e
            c_incl = _psum_incl(jnp.where(m, 1, 0), tmp_v, iota)
            rank = jnp.where(m, c_incl - 1, rank)
            tmp_v[...] = c_incl
            cnt = plsc.load_gather(tmp_v, [jnp.full((16,), 15, jnp.int32)])
            incr = incr + jnp.where(iota == e, cnt, 0)
        slot = plsc.load_gather(base_v, [ev]) + rank
        base_v[...] = base_v[...] + incr
        slot_v[b] = slot
        pos_v[pl.ds(g * 16, 16)] = slot + T
        tok_v[b] = w * TPW + rowi

    gsem = (sga, sgb)
    ssem = (ssa, ssb)
    compute(0, 0)
    gh = {0: pltpu.async_copy(x_hbm.at[tok_v.at[0]], rows_v.at[0], gsem[0])}
    sh = [None, None]
    for g in range(8):
        b = g & 1
        if g < 7:
            if sh[1 - b] is not None:
                sh[1 - b].wait()
            compute(g + 1, 1 - b)
            gh[g + 1] = pltpu.async_copy(x_hbm.at[tok_v.at[1 - b]],
                                         rows_v.at[1 - b], gsem[1 - b])
        gh[g].wait()
        sh[b] = pltpu.async_copy(rows_v.at[b], xs_hbm.at[slot_v.at[b]],
                                 ssem[b])
    sh[0].wait()
    sh[1].wait()

    pltpu.sync_copy(pos_v, pos_hbm.at[pl.ds(w * PPW, PPW)])


@jax.jit
def _dispatch(a1, a2, hist, x):
    mesh = plsc.VectorSubcoreMesh(core_axis_name="c", subcore_axis_name="s")
    f = functools.partial(
        pl.kernel,
        out_type=[
            jax.ShapeDtypeStruct((CAP, D), jnp.float32),
            jax.ShapeDtypeStruct((2 * T,), jnp.int32),
        ],
        mesh=mesh,
        scratch_types=[
            pltpu.VMEM((TPW,), jnp.int32),
            pltpu.VMEM((TPW,), jnp.int32),
            pltpu.VMEM((NW, 16), jnp.int32),
            pltpu.VMEM((2, 16, D), jnp.float32),
            pltpu.VMEM((2, 16), jnp.int32),
            pltpu.VMEM((2, 16), jnp.int32),
            pltpu.VMEM((PPW,), jnp.int32),
            pltpu.VMEM((16,), jnp.int32),
            pltpu.VMEM((16,), jnp.int32),
            pltpu.SemaphoreType.DMA,
            pltpu.SemaphoreType.DMA,
            pltpu.SemaphoreType.DMA,
            pltpu.SemaphoreType.DMA,
        ],
        compiler_params=pltpu.CompilerParams(needs_layout_passes=False),
    )(_dispatch_body)
    return f(a1, a2, hist, x)


def _gmm_body(sbex_ref, xs_ref, x_ref, wg_d, wu_d, wd_d, wg_s, wu_s, wd_s,
              out_ref):
    i = pl.program_id(0)
    e = sbex_ref[i]

    @pl.when(e == E_DYN)
    def _():
        x = x_ref[...]
        h = _silu(jnp.dot(x, wg_s[0], preferred_element_type=jnp.float32))
        h = h * jnp.dot(x, wu_s[0], preferred_element_type=jnp.float32)
        out_ref[...] = jnp.dot(h, wd_s[0], preferred_element_type=jnp.float32)

    @pl.when(e < E_DYN)
    def _():
        x = xs_ref[...]
        h = _silu(jnp.dot(x, wg_d[0], preferred_element_type=jnp.float32))
        h = h * jnp.dot(x, wu_d[0], preferred_element_type=jnp.float32)
        out_ref[...] = jnp.dot(h, wd_d[0], preferred_element_type=jnp.float32)


@jax.jit
def _gmm(sbex, xs, x, Wg_dyn, Wu_dyn, Wd_dyn, Wg_sh, Wu_sh, Wd_sh):
    clampe = lambda i, s: (jnp.minimum(s[i], E_DYN - 1), 0, 0)
    grid_spec = pltpu.PrefetchScalarGridSpec(
        num_scalar_prefetch=1,
        grid=(NBT,),
        in_specs=[
            pl.BlockSpec((M, D), lambda i, s: (jnp.maximum(i - NB_SH, 0), 0)),
            pl.BlockSpec((M, D), lambda i, s: (jnp.minimum(i, NB_SH - 1), 0)),
            pl.BlockSpec((1, D, DFF), clampe),
            pl.BlockSpec((1, D, DFF), clampe),
            pl.BlockSpec((1, DFF, D), clampe),
            pl.BlockSpec((1, D, DFF), lambda i, s: (0, 0, 0)),
            pl.BlockSpec((1, D, DFF), lambda i, s: (0, 0, 0)),
            pl.BlockSpec((1, DFF, D), lambda i, s: (0, 0, 0)),
        ],
        out_specs=pl.BlockSpec((M, D), lambda i, s: (i, 0)),
    )
    return pl.pallas_call(
        _gmm_body,
        grid_spec=grid_spec,
        out_shape=jax.ShapeDtypeStruct((ROWS_ALL, D), jnp.float32),
        compiler_params=pltpu.CompilerParams(
            dimension_semantics=("arbitrary",),
        ),
    )(sbex, xs, x, Wg_dyn, Wu_dyn, Wd_dyn, Wg_sh, Wu_sh, Wd_sh)


def _combine_body(all_hbm, pos_hbm, w1_hbm, w2_hbm, gf_hbm, out_hbm,
                  posg_v, w1_v, w2_v, gf_v, dyn_v, sh_v, out_v,
                  sga, sgb, soa, sob, shsem):
    w = lax.axis_index("s") * 2 + lax.axis_index("c")
    pltpu.sync_copy(w1_hbm.at[pl.ds(w * TPW, TPW)], w1_v)
    pltpu.sync_copy(w2_hbm.at[pl.ds(w * TPW, TPW)], w2_v)
    pltpu.sync_copy(gf_hbm.at[pl.ds(w * TPW, TPW)], gf_v)
    gsem = (sga, sgb)
    osem = (soa, sob)

    def fire(g, b):
        pltpu.sync_copy(pos_hbm.at[pl.ds(w * PPW + g * 16, 16)],
                        posg_v.at[b])
        return pltpu.async_copy(all_hbm.at[posg_v.at[b]], dyn_v.at[b],
                                gsem[b])

    handles = {0: fire(0, 0)}
    out_h = [None, None]
    for g in range(8):
        b = g % 2
        t0 = w * TPW + g * 8
        sh_h = pltpu.async_copy(all_hbm.at[pl.ds(t0, 8)], sh_v, shsem)
        if g < 7:
            handles[g + 1] = fire(g + 1, 1 - b)
        handles[g].wait()
        sh_h.wait()
        if out_h[0] is not None:
            out_h[0].wait()
        for j in range(8):
            w0 = w1_v[g * 8 + j]
            w1 = w2_v[g * 8 + j]
            gf = gf_v[g * 8 + j]

            def vbody(vo, _):
                for vi in range(8):
                    sl = pl.ds(vo * 128 + vi * 16, 16)
                    out_v[j, sl] = (w0 * dyn_v[b, 2 * j, sl]
                                    + w1 * dyn_v[b, 2 * j + 1, sl]
                                    + gf * sh_v[j, sl])
                return _

            lax.fori_loop(0, D // 128, vbody, None)
        out_h[0] = pltpu.async_copy(out_v, out_hbm.at[pl.ds(t0, 8)],
                                    osem[0])
    out_h[0].wait()


@jax.jit
def _combine(allout, pos, w1, w2, gf):
    mesh = plsc.VectorSubcoreMesh(core_axis_name="c", subcore_axis_name="s")
    f = functools.partial(
        pl.kernel,
        out_type=jax.ShapeDtypeStruct((T, D), jnp.float32),
        mesh=mesh,
        scratch_types=[
            pltpu.VMEM((2, 16), jnp.int32),
            pltpu.VMEM((TPW, 16), jnp.float32),
            pltpu.VMEM((TPW, 16), jnp.float32),
            pltpu.VMEM((TPW, 16), jnp.float32),
            pltpu.VMEM((2, 16, D), jnp.float32),
            pltpu.VMEM((8, D), jnp.float32),
            pltpu.VMEM((8, D), jnp.float32),
            pltpu.SemaphoreType.DMA,
            pltpu.SemaphoreType.DMA,
            pltpu.SemaphoreType.DMA,
            pltpu.SemaphoreType.DMA,
            pltpu.SemaphoreType.DMA,
        ],
        compiler_params=pltpu.CompilerParams(needs_layout_passes=False),
    )(_combine_body)
    return f(allout, pos, w1, w2, gf)


def kernel(hidden_states, W_router, Wg_dyn, Wu_dyn, Wd_dyn, Wg_sh, Wu_sh, Wd_sh):
    B, S, Dm = hidden_states.shape
    x = hidden_states.reshape(-1, Dm)
    a1, a2, w1, w2, gf, hist, sbex = _route(x, W_router)
    a1, a2 = a1.reshape(T), a2.reshape(T)
    sbex = sbex.reshape(64)
    xs, pos = _dispatch(a1, a2, hist, x)
    allout = _gmm(sbex, xs, x, Wg_dyn, Wu_dyn, Wd_dyn, Wg_sh, Wu_sh, Wd_sh)
    out = _combine(allout, pos, w1, w2, gf)
    return out.reshape(B, S, Dm)
